# baseline (speedup 1.0000x reference)
import jax
import jax.numpy as jnp
from jax import lax
from jax.experimental import pallas as pl
from jax.experimental.pallas import tpu as pltpu
from jax.experimental.pallas import tpu_sc as plsc

NC, NS = 2, 16
NW = NC * NS
CHUNK = 128
B, S, D = 4096, 50, 64
TOTAL = B * S
PER_W = TOTAL // NW
NCHUNK = PER_W // CHUNK


def _gather_body(idx_hbm, table_hbm, out_hbm, idx_v, rows_v, sem):
    wid = lax.axis_index("s") * NC + lax.axis_index("c")
    pltpu.sync_copy(idx_hbm.at[wid], idx_v)

    def step(j, carry):
        pltpu.async_copy(table_hbm.at[idx_v.at[j]], rows_v, sem).wait()
        pltpu.sync_copy(rows_v, out_hbm.at[wid, j])
        return carry

    lax.fori_loop(0, NCHUNK, step, 0)


_mesh = plsc.VectorSubcoreMesh(core_axis_name="c", subcore_axis_name="s")

_gather = pl.kernel(
    _gather_body,
    out_type=jax.ShapeDtypeStruct((NW, NCHUNK, CHUNK, D), jnp.float32),
    mesh=_mesh,
    scratch_types=[
        pltpu.VMEM((NCHUNK, CHUNK), jnp.int32),
        pltpu.VMEM((CHUNK, D), jnp.float32),
        pltpu.SemaphoreType.DMA,
    ],
    compiler_params=pltpu.CompilerParams(use_tc_tiling_on_sc=False),
)


def kernel(x, weight):
    idx = x.reshape(NW, NCHUNK, CHUNK).astype(jnp.int32)
    out = _gather(idx, weight)
    return out.reshape(B, S, D)


# speedup vs baseline: 1.0441x; 1.0441x over previous
import jax
import jax.numpy as jnp
from jax import lax
from jax.experimental import pallas as pl
from jax.experimental.pallas import tpu as pltpu
from jax.experimental.pallas import tpu_sc as plsc

NC, NS = 2, 16
NW = NC * NS
CHUNK = 128
B, S, D = 4096, 50, 64
TOTAL = B * S
PER_W = TOTAL // NW
NCHUNK = PER_W // CHUNK
NBUF = 8


def _gather_body(idx_hbm, table_hbm, out_hbm, idx_v, rows_v, gsem, wsem):
    wid = lax.axis_index("s") * NC + lax.axis_index("c")
    pltpu.sync_copy(idx_hbm.at[wid], idx_v)

    for s in range(NBUF):
        pltpu.async_copy(table_hbm.at[idx_v.at[s]], rows_v.at[s], gsem.at[s])

    def step(j, carry):
        s = j & (NBUF - 1)

        @pl.when(jnp.logical_and(j >= 1, j + NBUF - 1 < NCHUNK))
        def _():
            sp = (j - 1) & (NBUF - 1)
            pltpu.make_async_copy(
                rows_v.at[sp], out_hbm.at[wid, j - 1], wsem.at[sp]).wait()
            pltpu.async_copy(
                table_hbm.at[idx_v.at[j - 1 + NBUF]], rows_v.at[sp],
                gsem.at[sp])

        pltpu.make_async_copy(
            table_hbm.at[idx_v.at[s]], rows_v.at[s], gsem.at[s]).wait()
        pltpu.async_copy(rows_v.at[s], out_hbm.at[wid, j], wsem.at[s])
        return carry

    lax.fori_loop(0, NCHUNK, step, 0)

    for s in range(NBUF):
        j_last = max(j for j in range(NCHUNK) if j % NBUF == s)
        pltpu.make_async_copy(
            rows_v.at[s], out_hbm.at[wid, j_last], wsem.at[s]).wait()


_mesh = plsc.VectorSubcoreMesh(core_axis_name="c", subcore_axis_name="s")

_gather = pl.kernel(
    _gather_body,
    out_type=jax.ShapeDtypeStruct((NW, NCHUNK, CHUNK, D), jnp.float32),
    mesh=_mesh,
    scratch_types=[
        pltpu.VMEM((NCHUNK, CHUNK), jnp.int32),
        pltpu.VMEM((NBUF, CHUNK, D), jnp.float32),
        pltpu.SemaphoreType.DMA((NBUF,)),
        pltpu.SemaphoreType.DMA((NBUF,)),
    ],
    compiler_params=pltpu.CompilerParams(use_tc_tiling_on_sc=False),
)


def kernel(x, weight):
    idx = x.reshape(NW, NCHUNK, CHUNK).astype(jnp.int32)
    out = _gather(idx, weight)
    return out.reshape(B, S, D)


# speedup vs baseline: 1.1082x; 1.0613x over previous
import jax
import jax.numpy as jnp
from jax import lax
from jax.experimental import pallas as pl
from jax.experimental.pallas import tpu as pltpu
from jax.experimental.pallas import tpu_sc as plsc

NC, NS = 2, 16
NW = NC * NS
CHUNK = 128
B, S, D = 4096, 50, 64
DP = 128
TOTAL = B * S
PER_W = TOTAL // NW
NCHUNK = PER_W // CHUNK
NBUF = 4


def _gather_body(idx_hbm, table_hbm, out_hbm, idx_v, rows_v, gsem, wsem):
    wid = lax.axis_index("s") * NC + lax.axis_index("c")
    pltpu.sync_copy(idx_hbm.at[wid], idx_v)

    for s in range(NBUF):
        pltpu.async_copy(table_hbm.at[idx_v.at[s]], rows_v.at[s], gsem.at[s])

    def step(j, carry):
        s = j & (NBUF - 1)

        @pl.when(jnp.logical_and(j >= 1, j + NBUF - 1 < NCHUNK))
        def _():
            sp = (j - 1) & (NBUF - 1)
            pltpu.make_async_copy(
                rows_v.at[sp, :, pl.ds(0, D)], out_hbm.at[wid, j - 1],
                wsem.at[sp]).wait()
            pltpu.async_copy(
                table_hbm.at[idx_v.at[j - 1 + NBUF]], rows_v.at[sp],
                gsem.at[sp])

        pltpu.make_async_copy(
            table_hbm.at[idx_v.at[s]], rows_v.at[s], gsem.at[s]).wait()
        pltpu.async_copy(
            rows_v.at[s, :, pl.ds(0, D)], out_hbm.at[wid, j], wsem.at[s])
        return carry

    lax.fori_loop(0, NCHUNK, step, 0)

    for s in range(NBUF):
        j_last = max(j for j in range(NCHUNK) if j % NBUF == s)
        pltpu.make_async_copy(
            rows_v.at[s, :, pl.ds(0, D)], out_hbm.at[wid, j_last],
            wsem.at[s]).wait()


_mesh = plsc.VectorSubcoreMesh(core_axis_name="c", subcore_axis_name="s")

_gather = pl.kernel(
    _gather_body,
    out_type=jax.ShapeDtypeStruct((NW, NCHUNK, CHUNK, D), jnp.float32),
    mesh=_mesh,
    scratch_types=[
        pltpu.VMEM((NCHUNK, CHUNK), jnp.int32),
        pltpu.VMEM((NBUF, CHUNK, DP), jnp.float32),
        pltpu.SemaphoreType.DMA((NBUF,)),
        pltpu.SemaphoreType.DMA((NBUF,)),
    ],
    compiler_params=pltpu.CompilerParams(use_tc_tiling_on_sc=False),
)


def kernel(x, weight):
    idx = x.reshape(NW, NCHUNK, CHUNK).astype(jnp.int32)
    wt_wide = jnp.pad(weight, ((0, 0), (0, DP - D)))
    out = _gather(idx, wt_wide)
    return out.reshape(B, S, D)


# speedup vs baseline: 1.4933x; 1.3476x over previous
import jax
import jax.numpy as jnp
from jax import lax
from jax.experimental import pallas as pl
from jax.experimental.pallas import tpu as pltpu
from jax.experimental.pallas import tpu_sc as plsc

NC, NS = 2, 16
NW = NC * NS
CHUNK = 128
B, S, D = 4096, 50, 64
DP = 128
TOTAL = B * S
PER_W = TOTAL // NW
NCHUNK = PER_W // CHUNK
NBUF = 4


def _gather_body(idx_hbm, table_hbm, out_hbm, idx_v, rows_v, gsem, wsem):
    wid = lax.axis_index("s") * NC + lax.axis_index("c")
    pltpu.sync_copy(idx_hbm.at[wid], idx_v)

    for s in range(NBUF):
        pltpu.async_copy(table_hbm.at[idx_v.at[s]], rows_v.at[s], gsem.at[s])

    def step(j, carry):
        s = j & (NBUF - 1)

        @pl.when(jnp.logical_and(j >= 1, j + NBUF - 1 < NCHUNK))
        def _():
            sp = (j - 1) & (NBUF - 1)
            pltpu.make_async_copy(
                rows_v.at[sp, :, pl.ds(0, D)], out_hbm.at[wid, j - 1],
                wsem.at[sp]).wait()
            pltpu.async_copy(
                table_hbm.at[idx_v.at[j - 1 + NBUF]], rows_v.at[sp],
                gsem.at[sp])

        pltpu.make_async_copy(
            table_hbm.at[idx_v.at[s]], rows_v.at[s], gsem.at[s]).wait()
        pltpu.async_copy(
            rows_v.at[s, :, pl.ds(0, D)], out_hbm.at[wid, j], wsem.at[s])
        return carry

    lax.fori_loop(0, NCHUNK, step, 0)

    for s in range(NBUF):
        j_last = max(j for j in range(NCHUNK) if j % NBUF == s)
        pltpu.make_async_copy(
            rows_v.at[s, :, pl.ds(0, D)], out_hbm.at[wid, j_last],
            wsem.at[s]).wait()


BL = 4096
NBLK = (NUM_ROWS := 1000000, (1000000 + BL - 1) // BL)[1]


def _widen_body(in_ref, out_ref):
    out_ref[:, 0:D] = in_ref[...].T


_widen = pl.pallas_call(
    _widen_body,
    grid=(NBLK,),
    in_specs=[pl.BlockSpec((D, BL), lambda j: (0, j))],
    out_specs=pl.BlockSpec((BL, DP), lambda j: (j, 0)),
    out_shape=jax.ShapeDtypeStruct((NUM_ROWS, DP), jnp.float32),
    compiler_params=pltpu.CompilerParams(
        dimension_semantics=("arbitrary",)),
)

_mesh = plsc.VectorSubcoreMesh(core_axis_name="c", subcore_axis_name="s")

_gather = pl.kernel(
    _gather_body,
    out_type=jax.ShapeDtypeStruct((NW, NCHUNK, CHUNK, D), jnp.float32),
    mesh=_mesh,
    scratch_types=[
        pltpu.VMEM((NCHUNK, CHUNK), jnp.int32),
        pltpu.VMEM((NBUF, CHUNK, DP), jnp.float32),
        pltpu.SemaphoreType.DMA((NBUF,)),
        pltpu.SemaphoreType.DMA((NBUF,)),
    ],
    compiler_params=pltpu.CompilerParams(use_tc_tiling_on_sc=False),
)


def kernel(x, weight):
    idx = x.reshape(NW, NCHUNK, CHUNK).astype(jnp.int32)
    wt_wide = _widen(weight.T)
    out = _gather(idx, wt_wide)
    return out.reshape(B, S, D)


# speedup vs baseline: 1.7260x; 1.1558x over previous
import jax
import jax.numpy as jnp
from jax import lax
from jax.experimental import pallas as pl
from jax.experimental.pallas import tpu as pltpu
from jax.experimental.pallas import tpu_sc as plsc

NC, NS = 2, 16
NW = NC * NS
CHUNK = 128
B, S, D = 4096, 50, 64
DP = 128
TOTAL = B * S
PER_W = TOTAL // NW
NCHUNK = PER_W // CHUNK
NBUF = 4


def _gather_body(idx_hbm, table_hbm, out_hbm, idx_v, rows_v, gsem, wsem):
    wid = lax.axis_index("s") * NC + lax.axis_index("c")
    pltpu.sync_copy(idx_hbm.at[wid], idx_v)

    for s in range(NBUF):
        pltpu.async_copy(table_hbm.at[idx_v.at[s]], rows_v.at[s], gsem.at[s])

    def step(j, carry):
        s = j & (NBUF - 1)

        @pl.when(jnp.logical_and(j >= 1, j + NBUF - 1 < NCHUNK))
        def _():
            sp = (j - 1) & (NBUF - 1)
            pltpu.make_async_copy(
                rows_v.at[sp, :, pl.ds(0, D)], out_hbm.at[wid, j - 1],
                wsem.at[sp]).wait()
            pltpu.async_copy(
                table_hbm.at[idx_v.at[j - 1 + NBUF]], rows_v.at[sp],
                gsem.at[sp])

        pltpu.make_async_copy(
            table_hbm.at[idx_v.at[s]], rows_v.at[s], gsem.at[s]).wait()
        pltpu.async_copy(
            rows_v.at[s, :, pl.ds(0, D)], out_hbm.at[wid, j], wsem.at[s])
        return carry

    lax.fori_loop(0, NCHUNK, step, 0)

    for s in range(NBUF):
        j_last = max(j for j in range(NCHUNK) if j % NBUF == s)
        pltpu.make_async_copy(
            rows_v.at[s, :, pl.ds(0, D)], out_hbm.at[wid, j_last],
            wsem.at[s]).wait()


BL = 8192
NBLK = (NUM_ROWS := 1000000, (1000000 + BL - 1) // BL)[1]


def _widen_body(in_ref, out_ref):
    out_ref[:, 0:D] = in_ref[...].T


_widen = pl.pallas_call(
    _widen_body,
    grid=(NBLK,),
    in_specs=[pl.BlockSpec((D, BL), lambda j: (0, j))],
    out_specs=pl.BlockSpec((BL, DP), lambda j: (j, 0)),
    out_shape=jax.ShapeDtypeStruct((NUM_ROWS, DP), jnp.float32),
    compiler_params=pltpu.CompilerParams(
        dimension_semantics=("arbitrary",)),
)

_mesh = plsc.VectorSubcoreMesh(core_axis_name="c", subcore_axis_name="s")

_gather = pl.kernel(
    _gather_body,
    out_type=jax.ShapeDtypeStruct((NW, NCHUNK, CHUNK, D), jnp.float32),
    mesh=_mesh,
    scratch_types=[
        pltpu.VMEM((NCHUNK, CHUNK), jnp.int32),
        pltpu.VMEM((NBUF, CHUNK, DP), jnp.float32),
        pltpu.SemaphoreType.DMA((NBUF,)),
        pltpu.SemaphoreType.DMA((NBUF,)),
    ],
    compiler_params=pltpu.CompilerParams(use_tc_tiling_on_sc=False),
)


def kernel(x, weight):
    idx = x.reshape(NW, NCHUNK, CHUNK).astype(jnp.int32)
    wt_wide = _widen(weight.T)
    out = _gather(idx, wt_wide)
    return out.reshape(B, S, D)


# speedup vs baseline: 1.8024x; 1.0443x over previous
import jax
import jax.numpy as jnp
from jax import lax
from jax.experimental import pallas as pl
from jax.experimental.pallas import tpu as pltpu
from jax.experimental.pallas import tpu_sc as plsc

NC, NS = 2, 16
NW = NC * NS
CHUNK = 128
B, S, D = 4096, 50, 64
DP = 128
TOTAL = B * S
PER_W = TOTAL // NW
NCHUNK = PER_W // CHUNK
NBUF = 4


def _gather_body(idx_hbm, table_hbm, out_hbm, idx_v, rows_v, gsem, wsem):
    wid = lax.axis_index("s") * NC + lax.axis_index("c")
    pltpu.sync_copy(idx_hbm.at[wid], idx_v)

    for s in range(NBUF):
        pltpu.async_copy(table_hbm.at[idx_v.at[s]], rows_v.at[s], gsem.at[s])

    def step(j, carry):
        s = j & (NBUF - 1)

        @pl.when(jnp.logical_and(j >= 1, j + NBUF - 1 < NCHUNK))
        def _():
            sp = (j - 1) & (NBUF - 1)
            pltpu.make_async_copy(
                rows_v.at[sp, :, pl.ds(0, D)], out_hbm.at[wid, j - 1],
                wsem.at[sp]).wait()
            pltpu.async_copy(
                table_hbm.at[idx_v.at[j - 1 + NBUF]], rows_v.at[sp],
                gsem.at[sp])

        pltpu.make_async_copy(
            table_hbm.at[idx_v.at[s]], rows_v.at[s], gsem.at[s]).wait()
        pltpu.async_copy(
            rows_v.at[s, :, pl.ds(0, D)], out_hbm.at[wid, j], wsem.at[s])
        return carry

    lax.fori_loop(0, NCHUNK, step, 0)

    for s in range(NBUF):
        j_last = max(j for j in range(NCHUNK) if j % NBUF == s)
        pltpu.make_async_copy(
            rows_v.at[s, :, pl.ds(0, D)], out_hbm.at[wid, j_last],
            wsem.at[s]).wait()


BL = 16384
NBLK = (NUM_ROWS := 1000000, (1000000 + BL - 1) // BL)[1]


def _widen_body(in_ref, out_ref):
    out_ref[:, 0:D] = in_ref[...].T


_widen = pl.pallas_call(
    _widen_body,
    grid=(NBLK,),
    in_specs=[pl.BlockSpec((D, BL), lambda j: (0, j))],
    out_specs=pl.BlockSpec((BL, DP), lambda j: (j, 0)),
    out_shape=jax.ShapeDtypeStruct((NUM_ROWS, DP), jnp.float32),
    compiler_params=pltpu.CompilerParams(
        dimension_semantics=("arbitrary",)),
)

_mesh = plsc.VectorSubcoreMesh(core_axis_name="c", subcore_axis_name="s")

_gather = pl.kernel(
    _gather_body,
    out_type=jax.ShapeDtypeStruct((NW, NCHUNK, CHUNK, D), jnp.float32),
    mesh=_mesh,
    scratch_types=[
        pltpu.VMEM((NCHUNK, CHUNK), jnp.int32),
        pltpu.VMEM((NBUF, CHUNK, DP), jnp.float32),
        pltpu.SemaphoreType.DMA((NBUF,)),
        pltpu.SemaphoreType.DMA((NBUF,)),
    ],
    compiler_params=pltpu.CompilerParams(use_tc_tiling_on_sc=False),
)


def kernel(x, weight):
    idx = x.reshape(NW, NCHUNK, CHUNK).astype(jnp.int32)
    wt_wide = _widen(weight.T)
    out = _gather(idx, wt_wide)
    return out.reshape(B, S, D)


# speedup vs baseline: 1.8282x; 1.0143x over previous
import jax
import jax.numpy as jnp
from jax import lax
from jax.experimental import pallas as pl
from jax.experimental.pallas import tpu as pltpu
from jax.experimental.pallas import tpu_sc as plsc

NC, NS = 2, 16
NW = NC * NS
CHUNK = 128
B, S, D = 4096, 50, 64
DP = 128
TOTAL = B * S
PER_W = TOTAL // NW
NCHUNK = PER_W // CHUNK
NBUF = 4


def _gather_body(idx_hbm, table_hbm, out_hbm, idx_v, rows_v, gsem, wsem):
    wid = lax.axis_index("s") * NC + lax.axis_index("c")
    pltpu.sync_copy(idx_hbm.at[wid], idx_v)

    for s in range(NBUF):
        pltpu.async_copy(table_hbm.at[idx_v.at[s]], rows_v.at[s], gsem.at[s])

    def step(j, carry):
        s = j & (NBUF - 1)

        @pl.when(jnp.logical_and(j >= 1, j + NBUF - 1 < NCHUNK))
        def _():
            sp = (j - 1) & (NBUF - 1)
            pltpu.make_async_copy(
                rows_v.at[sp, :, pl.ds(0, D)], out_hbm.at[wid, j - 1],
                wsem.at[sp]).wait()
            pltpu.async_copy(
                table_hbm.at[idx_v.at[j - 1 + NBUF]], rows_v.at[sp],
                gsem.at[sp])

        pltpu.make_async_copy(
            table_hbm.at[idx_v.at[s]], rows_v.at[s], gsem.at[s]).wait()
        pltpu.async_copy(
            rows_v.at[s, :, pl.ds(0, D)], out_hbm.at[wid, j], wsem.at[s])
        return carry

    lax.fori_loop(0, NCHUNK, step, 0)

    for s in range(NBUF):
        j_last = max(j for j in range(NCHUNK) if j % NBUF == s)
        pltpu.make_async_copy(
            rows_v.at[s, :, pl.ds(0, D)], out_hbm.at[wid, j_last],
            wsem.at[s]).wait()


BL = 32768
NBLK = (NUM_ROWS := 1000000, (1000000 + BL - 1) // BL)[1]


def _widen_body(in_ref, out_ref):
    out_ref[:, 0:D] = in_ref[...].T


_widen = pl.pallas_call(
    _widen_body,
    grid=(NBLK,),
    in_specs=[pl.BlockSpec((D, BL), lambda j: (0, j))],
    out_specs=pl.BlockSpec((BL, DP), lambda j: (j, 0)),
    out_shape=jax.ShapeDtypeStruct((NUM_ROWS, DP), jnp.float32),
    compiler_params=pltpu.CompilerParams(
        dimension_semantics=("arbitrary",)),
)

_mesh = plsc.VectorSubcoreMesh(core_axis_name="c", subcore_axis_name="s")

_gather = pl.kernel(
    _gather_body,
    out_type=jax.ShapeDtypeStruct((NW, NCHUNK, CHUNK, D), jnp.float32),
    mesh=_mesh,
    scratch_types=[
        pltpu.VMEM((NCHUNK, CHUNK), jnp.int32),
        pltpu.VMEM((NBUF, CHUNK, DP), jnp.float32),
        pltpu.SemaphoreType.DMA((NBUF,)),
        pltpu.SemaphoreType.DMA((NBUF,)),
    ],
    compiler_params=pltpu.CompilerParams(use_tc_tiling_on_sc=False),
)


def kernel(x, weight):
    idx = x.reshape(NW, NCHUNK, CHUNK).astype(jnp.int32)
    wt_wide = _widen(weight.T)
    out = _gather(idx, wt_wide)
    return out.reshape(B, S, D)
